# baseline (speedup 1.0000x reference)
import functools

import jax
import jax.numpy as jnp
from jax import lax
from jax.experimental import pallas as pl
from jax.experimental.pallas import tpu as pltpu
from jax.experimental.pallas import tpu_sc as plsc

DROP_FRAC = 0.25
INT_MIN = -(2**31)


def _sortable_key(x):
    x = x + 0.0
    b = lax.bitcast_convert_type(x, jnp.int32)
    return b ^ (lax.shift_right_arithmetic(b, 31) & 0x7FFFFFFF)


def _sum_body(x_ref, o_ref):
    x = x_ref[0]
    o_ref[...] = jnp.sum(x, axis=1).reshape(1, 1, -1)


def _mask_body(k, s_ref, m_ref):
    s = s_ref[...][:, 0, :]
    nb, hw_n = s.shape
    key = _sortable_key(s)

    ones_col = jnp.ones((hw_n, 1), jnp.float32)

    def count(pred):
        return jax.lax.dot(pred.astype(jnp.float32), ones_col)

    ts = jnp.full((nb, 1), INT_MIN, jnp.int32)
    for bit in range(31, -1, -1):
        if bit == 31:
            cand = ts ^ INT_MIN
        else:
            cand = ts | (1 << bit)
        ts = jnp.where(count(key >= cand) >= k, cand, ts)

    gt = key > ts
    eq = key == ts
    need_eq = k - count(gt)

    hw = lax.broadcasted_iota(jnp.int32, (nb, hw_n), 1)
    m = jnp.zeros((nb, 1), jnp.int32)
    for bit in range(10, -1, -1):
        cand = m | (1 << bit)
        c = count(eq & (hw < cand))
        m = jnp.where(c <= need_eq, cand, m)

    zero = gt | (eq & (hw < m))
    mask = jnp.where(zero, 0.0, 1.0)
    m_ref[...] = mask[:, None, :]


def _make_sc_mask(k, B, hw):
    mesh = plsc.VectorSubcoreMesh(core_axis_name="c", subcore_axis_name="s")
    info = plsc.get_sparse_core_info()
    nc = info.num_cores
    nchunk = hw // 16

    @functools.partial(
        pl.kernel,
        mesh=mesh,
        compiler_params=pltpu.CompilerParams(needs_layout_passes=False),
        out_type=jax.ShapeDtypeStruct((B, hw), jnp.float32),
        scratch_types=[
            pltpu.VMEM((hw,), jnp.float32),
            pltpu.VMEM((hw,), jnp.int32),
            pltpu.VMEM((hw,), jnp.float32),
        ],
    )
    def fn(s_hbm, m_hbm, s_v, key_v, m_v):
        wid = lax.axis_index("s") * nc + lax.axis_index("c")
        pltpu.sync_copy(s_hbm.at[wid], s_v)

        def conv(i, carry):
            x = s_v[pl.ds(i * 16, 16)] + 0.0
            b = lax.bitcast_convert_type(x, jnp.int32)
            key_v[pl.ds(i * 16, 16)] = b ^ (
                lax.shift_right_arithmetic(b, 31) & 0x7FFFFFFF)
            return carry

        lax.fori_loop(0, nchunk, conv, jnp.int32(0))

        unroll = 16
        nouter = nchunk // unroll

        def count_ge(cand):
            def body(i, acc):
                for u in range(unroll):
                    ch = key_v[pl.ds((i * unroll + u) * 16, 16)]
                    acc = acc + jnp.where(ch >= cand, 1, 0).astype(jnp.int32)
                return acc
            acc = lax.fori_loop(0, nouter, body,
                                jnp.zeros((16,), jnp.int32))
            return jnp.sum(acc)

        ts = jnp.int32(INT_MIN)
        for bit in range(31, -1, -1):
            if bit == 31:
                cand = ts ^ INT_MIN
            else:
                cand = ts | (1 << bit)
            ts = jnp.where(count_ge(cand) >= k, cand, ts)

        def count_eq_below(ts_, m_):
            def body(i, acc):
                for u in range(unroll):
                    c = i * unroll + u
                    ch = key_v[pl.ds(c * 16, 16)]
                    pos = lax.broadcasted_iota(jnp.int32, (16,), 0) + c * 16
                    sel = (ch == ts_) & (pos < m_)
                    acc = acc + jnp.where(sel, 1, 0).astype(jnp.int32)
                return acc
            acc = lax.fori_loop(0, nouter, body,
                                jnp.zeros((16,), jnp.int32))
            return jnp.sum(acc)

        gt_cnt = count_ge(jnp.where(ts == 0x7FFFFFFF, ts, ts + 1))
        gt_cnt = jnp.where(ts == 0x7FFFFFFF, jnp.int32(0), gt_cnt)
        need_eq = k - gt_cnt

        mi = jnp.int32(0)
        for bit in range(10, -1, -1):
            cand = mi | (1 << bit)
            mi = jnp.where(count_eq_below(ts, cand) <= need_eq, cand, mi)

        def wr(i, carry):
            ch = key_v[pl.ds(i * 16, 16)]
            pos = lax.broadcasted_iota(jnp.int32, (16,), 0) + i * 16
            zero = (ch > ts) | ((ch == ts) & (pos < mi))
            m_v[pl.ds(i * 16, 16)] = jnp.where(zero, 0.0, 1.0)
            return carry

        lax.fori_loop(0, nchunk, wr, jnp.int32(0))
        pltpu.sync_copy(m_v, m_hbm.at[wid])

    return fn


def _mul_body(x_ref, m_ref, o_ref):
    m_col = m_ref[0].reshape(-1, 1)
    o_ref[0] = x_ref[0] * m_col


@jax.jit
def kernel(features):
    B, C, H, W = features.shape
    hw = H * W
    k = int(DROP_FRAC * hw)
    xt = jnp.transpose(features, (0, 2, 3, 1)).reshape(B, hw, C)

    sums = pl.pallas_call(
        _sum_body,
        grid=(B,),
        in_specs=[pl.BlockSpec((1, hw, C), lambda b: (b, 0, 0))],
        out_specs=pl.BlockSpec((1, 1, hw), lambda b: (b, 0, 0)),
        out_shape=jax.ShapeDtypeStruct((B, 1, hw), jnp.float32),
    )(xt)

    mask2d = _make_sc_mask(k, B, hw)(sums.reshape(B, hw))
    mask_rows = mask2d.reshape(B, 1, hw)

    out = pl.pallas_call(
        _mul_body,
        grid=(B,),
        in_specs=[
            pl.BlockSpec((1, hw, C), lambda b: (b, 0, 0)),
            pl.BlockSpec((1, 1, hw), lambda b: (b, 0, 0)),
        ],
        out_specs=pl.BlockSpec((1, hw, C), lambda b: (b, 0, 0)),
        out_shape=jax.ShapeDtypeStruct((B, hw, C), features.dtype),
    )(xt, mask_rows)

    return jnp.transpose(out.reshape(B, H, W, C), (0, 3, 1, 2))


# speedup vs baseline: 1.0192x; 1.0192x over previous
import functools

import jax
import jax.numpy as jnp
from jax import lax
from jax.experimental import pallas as pl
from jax.experimental.pallas import tpu as pltpu
from jax.experimental.pallas import tpu_sc as plsc

DROP_FRAC = 0.25
INT_MIN = -(2**31)


def _sum_body(x_ref, o_ref):
    x = x_ref[0]
    o_ref[...] = jnp.sum(x, axis=1).reshape(1, 1, -1)


def _make_sc_mask(k, B, hw):
    mesh = plsc.VectorSubcoreMesh(core_axis_name="c", subcore_axis_name="s")
    info = plsc.get_sparse_core_info()
    nc = info.num_cores
    nchunk = hw // 16

    @functools.partial(
        pl.kernel,
        mesh=mesh,
        compiler_params=pltpu.CompilerParams(needs_layout_passes=False),
        out_type=jax.ShapeDtypeStruct((B, hw), jnp.float32),
        scratch_types=[
            pltpu.VMEM((hw,), jnp.float32),
            pltpu.VMEM((hw,), jnp.int32),
            pltpu.VMEM((hw,), jnp.float32),
        ],
    )
    def fn(s_hbm, m_hbm, s_v, key_v, m_v):
        wid = lax.axis_index("s") * nc + lax.axis_index("c")
        pltpu.sync_copy(s_hbm.at[wid], s_v)

        def conv(i, carry):
            x = s_v[pl.ds(i * 16, 16)] + 0.0
            b = lax.bitcast_convert_type(x, jnp.int32)
            key_v[pl.ds(i * 16, 16)] = b ^ (
                lax.shift_right_arithmetic(b, 31) & 0x7FFFFFFF)
            return carry

        lax.fori_loop(0, nchunk, conv, jnp.int32(0))

        unroll = 8
        nouter = nchunk // unroll

        def count_ge(cand):
            def body(i, acc):
                for u in range(unroll):
                    ch = key_v[pl.ds((i * unroll + u) * 16, 16)]
                    acc = acc + jnp.where(ch >= cand, 1, 0).astype(jnp.int32)
                return acc
            acc = lax.fori_loop(0, nouter, body,
                                jnp.zeros((16,), jnp.int32))
            return jnp.sum(acc)

        ts = jnp.int32(INT_MIN)
        for bit in range(31, -1, -1):
            if bit == 31:
                cand = ts ^ INT_MIN
            else:
                cand = ts | (1 << bit)
            ts = jnp.where(count_ge(cand) >= k, cand, ts)

        def count_eq_below(ts_, m_):
            def body(i, acc):
                for u in range(unroll):
                    c = i * unroll + u
                    ch = key_v[pl.ds(c * 16, 16)]
                    pos = lax.broadcasted_iota(jnp.int32, (16,), 0) + c * 16
                    sel = (ch == ts_) & (pos < m_)
                    acc = acc + jnp.where(sel, 1, 0).astype(jnp.int32)
                return acc
            acc = lax.fori_loop(0, nouter, body,
                                jnp.zeros((16,), jnp.int32))
            return jnp.sum(acc)

        gt_cnt = count_ge(jnp.where(ts == 0x7FFFFFFF, ts, ts + 1))
        gt_cnt = jnp.where(ts == 0x7FFFFFFF, jnp.int32(0), gt_cnt)
        need_eq = k - gt_cnt

        mi = jnp.int32(0)
        for bit in range(10, -1, -1):
            cand = mi | (1 << bit)
            mi = jnp.where(count_eq_below(ts, cand) <= need_eq, cand, mi)

        def wr(i, carry):
            ch = key_v[pl.ds(i * 16, 16)]
            pos = lax.broadcasted_iota(jnp.int32, (16,), 0) + i * 16
            zero = (ch > ts) | ((ch == ts) & (pos < mi))
            m_v[pl.ds(i * 16, 16)] = jnp.where(zero, 0.0, 1.0)
            return carry

        lax.fori_loop(0, nchunk, wr, jnp.int32(0))
        pltpu.sync_copy(m_v, m_hbm.at[wid])

    return fn


def _mul_body(x_ref, m_ref, o_ref):
    m_col = m_ref[0].reshape(-1, 1)
    o_ref[0] = x_ref[0] * m_col


@jax.jit
def kernel(features):
    B, C, H, W = features.shape
    hw = H * W
    k = int(DROP_FRAC * hw)
    xt = jnp.transpose(features, (0, 2, 3, 1)).reshape(B, hw, C)

    sums = pl.pallas_call(
        _sum_body,
        grid=(B,),
        in_specs=[pl.BlockSpec((1, hw, C), lambda b: (b, 0, 0))],
        out_specs=pl.BlockSpec((1, 1, hw), lambda b: (b, 0, 0)),
        out_shape=jax.ShapeDtypeStruct((B, 1, hw), jnp.float32),
    )(xt)

    mask2d = _make_sc_mask(k, B, hw)(sums.reshape(B, hw))
    mask_rows = mask2d.reshape(B, 1, hw)

    out = pl.pallas_call(
        _mul_body,
        grid=(B,),
        in_specs=[
            pl.BlockSpec((1, hw, C), lambda b: (b, 0, 0)),
            pl.BlockSpec((1, 1, hw), lambda b: (b, 0, 0)),
        ],
        out_specs=pl.BlockSpec((1, hw, C), lambda b: (b, 0, 0)),
        out_shape=jax.ShapeDtypeStruct((B, hw, C), features.dtype),
    )(xt, mask_rows)

    return jnp.transpose(out.reshape(B, H, W, C), (0, 3, 1, 2))
